# baseline (speedup 1.0000x reference)
import functools

import jax
import jax.numpy as jnp
from jax import lax
from jax.experimental import pallas as pl
from jax.experimental.pallas import tpu as pltpu
from jax.experimental.pallas import tpu_sc as plsc

_VOCAB = 30522
_D = 128
_B = 16384
_CH = 128


@functools.lru_cache(maxsize=1)
def _build():
    info = plsc.get_sparse_core_info()
    nc, ns = info.num_cores, info.num_subcores
    nw = nc * ns
    b_per_w = _B // nw
    nch = b_per_w // _CH
    units = 2 * nch
    nbuf = 7
    mesh = plsc.VectorSubcoreMesh(core_axis_name="c", subcore_axis_name="s")
    out_sds = jax.ShapeDtypeStruct((_B, _D), jnp.float32)

    @functools.partial(
        pl.kernel,
        mesh=mesh,
        out_type=[out_sds, out_sds],
        scratch_types=[
            pltpu.VMEM((units, _CH), jnp.int32),
            pltpu.VMEM((nbuf * _CH, _D), jnp.float32),
        ] + [pltpu.SemaphoreType.DMA] * (2 * nbuf + units),
    )
    def sc_gather2(go_idx, rel_idx, go_tab, rel_tab, go_out, rel_out,
                   idx_v, rows_v, *sems):
        gsem = sems[:nbuf]
        osem = sems[nbuf:2 * nbuf]
        isem = sems[2 * nbuf:]
        wid = lax.axis_index("s") * nc + lax.axis_index("c")

        idxs = [go_idx] * nch + [rel_idx] * nch
        tabs = [go_tab] * nch + [rel_tab] * nch
        outs = [go_out] * nch + [rel_out] * nch

        def buf(u):
            return rows_v.at[pl.ds((u % nbuf) * _CH, _CH)]

        def out_slice(u):
            return outs[u].at[pl.ds(wid * b_per_w + (u % nch) * _CH, _CH)]

        icp = [
            pltpu.async_copy(idxs[u].at[wid, u % nch], idx_v.at[u], isem[u])
            for u in range(units)
        ]

        gcp = [None] * units
        ocp = [None] * units
        for u in range(min(nbuf, units)):
            icp[u].wait()
            gcp[u] = pltpu.async_copy(
                tabs[u].at[idx_v.at[u]], buf(u), gsem[u % nbuf])
        for u in range(units):
            gcp[u].wait()
            ocp[u] = pltpu.async_copy(buf(u), out_slice(u), osem[u % nbuf])
            refire = u + nbuf
            if refire < units:
                ocp[u].wait()
                icp[refire].wait()
                gcp[refire] = pltpu.async_copy(
                    tabs[refire].at[idx_v.at[refire]], buf(refire),
                    gsem[refire % nbuf])
        for u in range(max(0, units - nbuf), units):
            ocp[u].wait()

    return sc_gather2, nw, nch


def kernel(go_inputs, relation_ids, go_table, rel_table):
    k, nw, nch = _build()
    go_idx = go_inputs.reshape(nw, nch, _CH)
    rel_idx = relation_ids.reshape(nw, nch, _CH)
    entity_embed, relation_embed = k(go_idx, rel_idx, go_table, rel_table)
    return (entity_embed, relation_embed)


# speedup vs baseline: 1.0175x; 1.0175x over previous
import functools

import jax
import jax.numpy as jnp
from jax import lax
from jax.experimental import pallas as pl
from jax.experimental.pallas import tpu as pltpu
from jax.experimental.pallas import tpu_sc as plsc

_VOCAB = 30522
_D = 128
_B = 16384
_CH = 128


@functools.lru_cache(maxsize=1)
def _build():
    info = plsc.get_sparse_core_info()
    nc, ns = info.num_cores, info.num_subcores
    nw = nc * ns
    b_per_w = _B // nw
    nch = b_per_w // _CH
    units = 2 * nch
    nbuf = 7
    mesh = plsc.VectorSubcoreMesh(core_axis_name="c", subcore_axis_name="s")
    out_sds = jax.ShapeDtypeStruct((_B, _D), jnp.float32)

    @functools.partial(
        pl.kernel,
        mesh=mesh,
        out_type=[out_sds, out_sds],
        scratch_types=[
            pltpu.VMEM((units, _CH), jnp.int32),
            pltpu.VMEM((nbuf * _CH, _D), jnp.float32),
        ] + [pltpu.SemaphoreType.DMA] * (2 * nbuf + 2),
    )
    def sc_gather2(go_idx, rel_idx, go_tab, rel_tab, go_out, rel_out,
                   idx_v, rows_v, *sems):
        gsem, osem, isem = sems[:nbuf], sems[nbuf:2 * nbuf], sems[2 * nbuf:]
        wid = lax.axis_index("s") * nc + lax.axis_index("c")

        icp0 = pltpu.async_copy(go_idx.at[wid], idx_v.at[pl.ds(0, nch)],
                                isem[0])
        icp1 = pltpu.async_copy(rel_idx.at[wid], idx_v.at[pl.ds(nch, nch)],
                                isem[1])

        tabs = [go_tab] * nch + [rel_tab] * nch
        outs = [go_out] * nch + [rel_out] * nch

        def buf(u):
            return rows_v.at[pl.ds((u % nbuf) * _CH, _CH)]

        def out_slice(u):
            return outs[u].at[pl.ds(wid * b_per_w + (u % nch) * _CH, _CH)]

        gcp = [None] * units
        ocp = [None] * units
        icp0.wait()
        for u in range(min(nbuf, units)):
            if u == nch:
                icp1.wait()
            gcp[u] = pltpu.async_copy(
                tabs[u].at[idx_v.at[u]], buf(u), gsem[u % nbuf])
        for u in range(units):
            gcp[u].wait()
            ocp[u] = pltpu.async_copy(buf(u), out_slice(u), osem[u % nbuf])
            refire = u + nbuf
            if refire < units:
                ocp[u].wait()
                gcp[refire] = pltpu.async_copy(
                    tabs[refire].at[idx_v.at[refire]], buf(refire),
                    gsem[refire % nbuf])
        for u in range(max(0, units - nbuf), units):
            ocp[u].wait()

    return sc_gather2, nw, nch


def kernel(go_inputs, relation_ids, go_table, rel_table):
    k, nw, nch = _build()
    go_idx = go_inputs.reshape(nw, nch, _CH)
    rel_idx = relation_ids.reshape(nw, nch, _CH)
    entity_embed, relation_embed = k(go_idx, rel_idx, go_table, rel_table)
    return (entity_embed, relation_embed)
